# baseline (speedup 1.0000x reference)
import functools

import jax
import jax.numpy as jnp
from jax import lax
from jax.experimental import pallas as pl
from jax.experimental.pallas import tpu as pltpu
from jax.experimental.pallas import tpu_sc as plsc

N_BASIS = 8
CUTOFF = 10.0
AVG_NEIGH = 16.0

NC = 2
NS = 16
NW = NC * NS
LANE = 128
CHR = 40

F32 = jnp.float32
I32 = jnp.int32


def _mesh():
    return plsc.VectorSubcoreMesh(core_axis_name="c", subcore_axis_name="s")


@functools.cache
def _make_gather(total_rows: int, table_rows: int):
    n_chunks = total_rows // (NW * CHR)
    assert total_rows == NW * CHR * n_chunks
    ce = CHR * LANE

    def body(table, idx2d, out, idxbuf, ebuf, sem):
        wid = lax.axis_index("s") * NC + lax.axis_index("c")
        for ch in range(n_chunks):
            row0 = wid * (CHR * n_chunks) + ch * CHR
            pltpu.sync_copy(idx2d.at[pl.ds(row0, CHR)], idxbuf)

            def fire(j, carry):
                pltpu.async_copy(table.at[idxbuf.at[j]],
                                 ebuf.at[pl.ds(j * LANE, LANE)], sem)
                return carry

            lax.fori_loop(0, CHR, fire, 0)
            pltpu.make_async_copy(table.at[pl.ds(0, ce)], ebuf, sem).wait()
            pltpu.sync_copy(ebuf, out.at[pl.ds(row0 * LANE, ce)])

    return pl.kernel(
        body,
        out_type=jax.ShapeDtypeStruct((total_rows * LANE, 8), F32),
        mesh=_mesh(),
        compiler_params=pltpu.CompilerParams(use_tc_tiling_on_sc=False),
        scratch_types=[
            pltpu.VMEM((CHR, LANE), I32),
            pltpu.VMEM((ce, 8), F32),
            pltpu.SemaphoreType.DMA,
        ],
    )


CHS = 20


@functools.cache
def _make_scatter(total_rows: int, np_rows: int):
    n_chunks = total_rows // (NW * CHS)
    assert total_rows == NW * CHS * n_chunks
    ce = CHS * LANE
    rz = np_rows // NS
    assert rz * NS == np_rows and rz % 8 == 0

    def body(edge, idx2d, zeros_hbm, out, idxbuf, ebuf, acc, sem):
        c = lax.axis_index("c")
        s = lax.axis_index("s")
        wid = s * NC + c
        pltpu.sync_copy(zeros_hbm.at[pl.ds(s * rz, rz)], acc.at[pl.ds(s * rz, rz)])
        plsc.subcore_barrier()
        for ch in range(n_chunks):
            row0 = wid * (CHS * n_chunks) + ch * CHS
            pltpu.sync_copy(idx2d.at[pl.ds(row0, CHS)], idxbuf)
            pltpu.sync_copy(edge.at[pl.ds(row0 * LANE, ce)], ebuf)

            def fire(j, carry):
                pltpu.async_copy(ebuf.at[pl.ds(j * LANE, LANE)],
                                 acc.at[idxbuf.at[j]], sem, add=True)
                return carry

            lax.fori_loop(0, CHS, fire, 0)
            pltpu.make_async_copy(edge.at[pl.ds(0, ce)], ebuf, sem).wait()
        plsc.subcore_barrier()
        pltpu.sync_copy(acc.at[pl.ds(s * rz, rz)], out.at[c, pl.ds(s * rz, rz)])

    return pl.kernel(
        body,
        out_type=jax.ShapeDtypeStruct((NC, np_rows, 16), F32),
        mesh=_mesh(),
        compiler_params=pltpu.CompilerParams(use_tc_tiling_on_sc=False),
        scratch_types=[
            pltpu.VMEM((CHS, LANE), I32),
            pltpu.VMEM((ce, 16), F32),
            pltpu.VMEM_SHARED((np_rows, 16), F32),
            pltpu.SemaphoreType.DMA,
        ],
    )


def _swish(x):
    return x / (1.0 + jnp.exp(-x))


def _sinpi(a):
    n = jnp.floor(a)
    r = a - n
    s = jnp.minimum(r, 1.0 - r)
    half = n * 0.5
    sgn = 1.0 - 4.0 * (half - jnp.floor(half))
    t = s * jnp.pi
    q = t * t
    p = t * (1.0 + q * (-1.0 / 6.0 + q * (1.0 / 120.0
             + q * (-1.0 / 5040.0 + q * (1.0 / 362880.0)))))
    return sgn * p


def _edge_math_t(ps, pr, fm, w0, b0c, w1, w2, wup, valid):
    dn_t = (((0,), (0,)), ((), ()))
    vec = pr - ps
    len2 = jnp.sum(vec * vec, axis=0, keepdims=True)
    length = jnp.sqrt(len2)
    u = length / CUTOFF
    safe = jnp.where(u == 0.0, 1.0, u)
    k = (lax.broadcasted_iota(I32, (N_BASIS, 1), 0) + 1).astype(F32)
    bess = jnp.sqrt(2.0) * _sinpi(u * k) / safe
    u5 = u * u * u * u * u
    env = 1.0 - 21.0 * u5 + 35.0 * u5 * u - 15.0 * u5 * u * u
    env = jnp.where(u < 1.0, env, 0.0)
    basis = jnp.where(length == 0.0, 0.0, bess * env)

    h = _swish(lax.dot_general(w0, basis, dn_t,
                               preferred_element_type=F32) + b0c)
    h = _swish(lax.dot_general(w1, h, dn_t, preferred_element_type=F32))
    mix = lax.dot_general(w2, h, dn_t, preferred_element_type=F32)

    nh = vec / (length + 1e-9)
    m0 = fm[0:3] * wup[0] + fm[3:6] * wup[2]
    m1 = fm[0:3] * wup[1] + fm[3:6] * wup[3]
    d0 = jnp.sum(m0 * nh, axis=0, keepdims=True)
    d1 = jnp.sum(m1 * nh, axis=0, keepdims=True)
    sqrt3 = jnp.sqrt(3.0)
    sqrt75 = jnp.sqrt(7.5)
    s0 = sqrt3 * d0 * mix[0:1]
    s1 = sqrt3 * d1 * mix[1:2]
    vm0 = m0 * mix[2:3]
    vm1 = m1 * mix[3:4]
    t0 = sqrt75 * (nh * d0 - m0 * (1.0 / 3.0)) * mix[4:5]
    t1 = sqrt75 * (nh * d1 - m1 * (1.0 / 3.0)) * mix[5:6]
    zz = jnp.zeros((2, s0.shape[1]), F32)
    out = jnp.concatenate([s0, s1, vm0, vm1, t0, t1, zz], axis=0)
    return jnp.where(valid, out, 0.0)


def _node_math(agg_a, agg_b, fp, wdv, wds, wsc):
    a = [agg_a[:, 2:5], agg_a[:, 5:8], agg_b[:, 0:3], agg_b[:, 3:6]]
    f0, f1 = fp[:, 0:3], fp[:, 3:6]
    g = []
    for kk in range(2):
        v = (a[0] * wdv[0][kk] + a[1] * wdv[1][kk]
             + a[2] * wdv[2][kk] + a[3] * wdv[3][kk]
             + f0 * wsc[0][kk] + f1 * wsc[1][kk])
        s = agg_a[:, 0:1] * wds[0][kk] + agg_a[:, 1:2] * wds[1][kk]
        g.append(v * _swish(s))
    zz = jnp.zeros_like(g[0][:, 0:1])
    gated = jnp.concatenate([g[0], g[1], zz, zz], axis=1)
    ss0 = jnp.sum(g[0] * g[0])
    ss1 = jnp.sum(g[1] * g[1])
    return gated, (ss0, ss1)


BE = 4096
BN = 2000


def _edge_kernel_call(posg, fmsg, w0, b0c, w1, w2, wup22, e_real, es_pad):
    nblk = es_pad // BE

    def body(wup_ref, ps_ref, pr_ref, fm_ref, w0_ref, b0_ref, w1_ref, w2_ref,
             out_ref):
        i = pl.program_id(0)
        cols = i * BE + lax.broadcasted_iota(I32, (1, BE), 1)
        valid = cols < e_real
        wup = (wup_ref[0, 0], wup_ref[0, 1], wup_ref[1, 0], wup_ref[1, 1])
        psT = jnp.transpose(ps_ref[...])[0:3]
        prT = jnp.transpose(pr_ref[...])[0:3]
        fmT = jnp.transpose(fm_ref[...])[0:6]
        outT = _edge_math_t(psT, prT, fmT,
                            w0_ref[...], b0_ref[...], w1_ref[...],
                            w2_ref[...], wup, valid)
        out_ref[...] = jnp.transpose(outT)

    return pl.pallas_call(
        body,
        grid=(nblk,),
        in_specs=[
            pl.BlockSpec(memory_space=pltpu.SMEM),
            pl.BlockSpec((BE, 8), lambda i: (i, 0)),
            pl.BlockSpec((BE, 8), lambda i, off=nblk: (i + off, 0)),
            pl.BlockSpec((BE, 8), lambda i: (i, 0)),
            pl.BlockSpec((8, 64), lambda i: (0, 0)),
            pl.BlockSpec((64, 1), lambda i: (0, 0)),
            pl.BlockSpec((64, 64), lambda i: (0, 0)),
            pl.BlockSpec((64, 8), lambda i: (0, 0)),
        ],
        out_specs=pl.BlockSpec((BE, 16), lambda i: (i, 0)),
        out_shape=jax.ShapeDtypeStruct((es_pad, 16), F32),
    )(wup22, posg, posg, fmsg, w0, b0c, w1, w2)


def _node_kernel_call(acc, featp, wdv42, wds22, wsc22, n_nodes):
    nblk = n_nodes // BN

    def body(wdv_ref, wds_ref, wsc_ref, acc_ref, fp_ref, g_ref,
             ssq_ref):
        i = pl.program_id(0)
        agg = acc_ref[0] + acc_ref[1]
        agg_a = agg[:, 0:8]
        agg_b = agg[:, 8:16]
        wdv = tuple((wdv_ref[r, 0], wdv_ref[r, 1]) for r in range(4))
        wds = tuple((wds_ref[r, 0], wds_ref[r, 1]) for r in range(2))
        wsc = tuple((wsc_ref[r, 0], wsc_ref[r, 1]) for r in range(2))
        gated, (ss0, ss1) = _node_math(agg_a, agg_b, fp_ref[...], wdv, wds, wsc)
        g_ref[...] = gated
        row = jnp.concatenate(
            [ss0.reshape(1, 1), ss1.reshape(1, 1), jnp.zeros((1, 6), F32)],
            axis=1)

        @pl.when(i == 0)
        def _():
            ssq_ref[...] = row

        @pl.when(i != 0)
        def _():
            ssq_ref[...] = ssq_ref[...] + row

    return pl.pallas_call(
        body,
        grid=(nblk,),
        in_specs=[
            pl.BlockSpec(memory_space=pltpu.SMEM),
            pl.BlockSpec(memory_space=pltpu.SMEM),
            pl.BlockSpec(memory_space=pltpu.SMEM),
            pl.BlockSpec((2, BN, 16), lambda i: (0, i, 0)),
            pl.BlockSpec((BN, 8), lambda i: (i, 0)),
        ],
        out_specs=[
            pl.BlockSpec((BN, 8), lambda i: (i, 0)),
            pl.BlockSpec((1, 8), lambda i: (0, 0)),
        ],
        out_shape=[
            jax.ShapeDtypeStruct((n_nodes, 8), F32),
            jax.ShapeDtypeStruct((1, 8), F32),
        ],
    )(wdv42, wds22, wsc22, acc, featp)


def _final_kernel_call(nodes, gated, irms12, n_nodes):
    nblk = n_nodes // BN

    def body(ir_ref, nd_ref, g_ref, out_ref):
        disp = nd_ref[:, 0:3] + g_ref[:, 0:3] * ir_ref[0, 0]
        vel = g_ref[:, 3:6] * ir_ref[0, 1]
        out_ref[...] = jnp.concatenate([disp, vel], axis=1)

    return pl.pallas_call(
        body,
        grid=(nblk,),
        in_specs=[
            pl.BlockSpec(memory_space=pltpu.SMEM),
            pl.BlockSpec((BN, 6), lambda i: (i, 0)),
            pl.BlockSpec((BN, 8), lambda i: (i, 0)),
        ],
        out_specs=pl.BlockSpec((BN, 6), lambda i: (i, 0)),
        out_shape=jax.ShapeDtypeStruct((n_nodes, 6), F32),
    )(irms12, nodes, gated)


def _prep_weights(p, m, gvec, rms_prev):
    inv_r = 1.0 / rms_prev
    wsc = jnp.zeros((2, 2), F32).at[:m].set(p['W_sc'] * inv_r[:m, None])
    wup = jnp.zeros((2, 2), F32).at[:m, :m].set(p['W_up'] * inv_r[:m, None])
    w0 = p['W0'][:N_BASIS]
    b0 = (gvec.reshape(1, -1) @ p['W0'][N_BASIS:]).astype(F32).reshape(64, 1)
    w1 = p['W1']
    w2 = jnp.zeros((64, 8), F32)
    if m == 2:
        w2 = w2.at[:, 0:6].set(p['W2'])
    else:
        w2 = (w2.at[:, 0].set(p['W2'][:, 0])
                .at[:, 2].set(p['W2'][:, 1])
                .at[:, 4].set(p['W2'][:, 2]))
    inv_an = 1.0 / jnp.sqrt(AVG_NEIGH)
    wdvs = p['Wd_v'] * inv_an
    if m == 2:
        wdv = wdvs
    else:
        wdv = jnp.zeros((4, 2), F32).at[0].set(wdvs[0]).at[2].set(wdvs[1])
    wds = jnp.zeros((2, 2), F32).at[:m].set(p['Wd_s'] * inv_an)
    return wsc, wup, w0, b0, w1, w2, wdv, wds


def kernel(nodes, globals, params, senders, receivers):
    n = nodes.shape[0]
    e = senders.shape[0]
    assert n % BN == 0

    unit = NW * CHR * LANE
    es = ((e + unit - 1) // unit) * unit
    np_rows = ((n + NS * 8 - 1) // (NS * 8)) * (NS * 8)

    pad = jnp.zeros((es - e,), I32)
    send_p = jnp.concatenate([senders, pad])
    recv_p = jnp.concatenate([receivers, pad])
    sr2 = jnp.concatenate([send_p, recv_p]).reshape(-1, LANE)
    send2 = send_p.reshape(-1, LANE)
    recv2 = recv_p.reshape(-1, LANE)

    pos_tab = jnp.pad(nodes[:, 0:3], ((0, 0), (0, 5)))
    feat = jnp.pad(nodes[:, 3:6], ((0, 0), (0, 5)))
    zeros_acc = jnp.zeros((np_rows, 16), F32)
    gvec = globals.astype(F32)

    posg = _make_gather(2 * es // LANE, n)(pos_tab, sr2)

    rms_prev = jnp.ones((2,), F32)
    for li, p in enumerate(params):
        m = 1 if li == 0 else 2
        wsc, wup, w0, b0, w1, w2, wdv, wds = _prep_weights(p, m, gvec, rms_prev)
        fmsg = _make_gather(es // LANE, n)(feat, send2)
        eout = _edge_kernel_call(posg, fmsg, w0, b0, w1, w2, wup, e, es)
        acc = _make_scatter(es // LANE, np_rows)(eout, recv2, zeros_acc)
        gated, ssq = _node_kernel_call(acc, feat, wdv, wds, wsc, n)
        rms_prev = jnp.sqrt(ssq[0, 0:2] / n) + 1e-5
        feat = gated

    irms = (1.0 / rms_prev).reshape(1, 2)
    return _final_kernel_call(nodes, feat, irms, n)


# speedup vs baseline: 1.0486x; 1.0486x over previous
import functools

import jax
import jax.numpy as jnp
from jax import lax
from jax.experimental import pallas as pl
from jax.experimental.pallas import tpu as pltpu
from jax.experimental.pallas import tpu_sc as plsc

N_BASIS = 8
CUTOFF = 10.0
AVG_NEIGH = 16.0

NC = 2
NS = 16
NW = NC * NS
LANE = 128
CHR = 40

F32 = jnp.float32
I32 = jnp.int32


def _mesh():
    return plsc.VectorSubcoreMesh(core_axis_name="c", subcore_axis_name="s")


@functools.cache
def _make_gather(total_rows: int, table_rows: int):
    n_chunks = total_rows // (NW * CHR)
    assert total_rows == NW * CHR * n_chunks
    ce = CHR * LANE

    def body(table, idx2d, out, idxbuf, ebuf, sem):
        wid = lax.axis_index("s") * NC + lax.axis_index("c")
        for ch in range(n_chunks):
            row0 = wid * (CHR * n_chunks) + ch * CHR
            pltpu.sync_copy(idx2d.at[pl.ds(row0, CHR)], idxbuf)

            def fire(j, carry):
                pltpu.async_copy(table.at[idxbuf.at[j]],
                                 ebuf.at[pl.ds(j * LANE, LANE)], sem)
                return carry

            lax.fori_loop(0, CHR, fire, 0)
            pltpu.make_async_copy(table.at[pl.ds(0, ce)], ebuf, sem).wait()
            pltpu.sync_copy(ebuf, out.at[pl.ds(row0 * LANE, ce)])

    return pl.kernel(
        body,
        out_type=jax.ShapeDtypeStruct((total_rows * LANE, 8), F32),
        mesh=_mesh(),
        compiler_params=pltpu.CompilerParams(use_tc_tiling_on_sc=False),
        scratch_types=[
            pltpu.VMEM((CHR, LANE), I32),
            pltpu.VMEM((ce, 8), F32),
            pltpu.SemaphoreType.DMA,
        ],
    )


CHS = 20


@functools.cache
def _make_scatter(total_rows: int, np_rows: int):
    n_chunks = total_rows // (NW * CHS)
    assert total_rows == NW * CHS * n_chunks
    ce = CHS * LANE
    rz = np_rows // NS
    assert rz * NS == np_rows and rz % 8 == 0

    def body(edge, idx2d, zeros_hbm, out, idxbuf, ebuf, acc, sem):
        c = lax.axis_index("c")
        s = lax.axis_index("s")
        wid = s * NC + c
        pltpu.sync_copy(zeros_hbm.at[pl.ds(s * rz, rz)], acc.at[pl.ds(s * rz, rz)])
        plsc.subcore_barrier()
        for ch in range(n_chunks):
            row0 = wid * (CHS * n_chunks) + ch * CHS
            pltpu.sync_copy(idx2d.at[pl.ds(row0, CHS)], idxbuf)
            pltpu.sync_copy(edge.at[pl.ds(row0 * LANE, ce)], ebuf)

            def fire(j, carry):
                pltpu.async_copy(ebuf.at[pl.ds(j * LANE, LANE)],
                                 acc.at[idxbuf.at[j]], sem, add=True)
                return carry

            lax.fori_loop(0, CHS, fire, 0)
            pltpu.make_async_copy(edge.at[pl.ds(0, ce)], ebuf, sem).wait()
        plsc.subcore_barrier()
        pltpu.sync_copy(acc.at[pl.ds(s * rz, rz)], out.at[c, pl.ds(s * rz, rz)])

    return pl.kernel(
        body,
        out_type=jax.ShapeDtypeStruct((NC, np_rows, 16), F32),
        mesh=_mesh(),
        compiler_params=pltpu.CompilerParams(use_tc_tiling_on_sc=False),
        scratch_types=[
            pltpu.VMEM((CHS, LANE), I32),
            pltpu.VMEM((ce, 16), F32),
            pltpu.VMEM_SHARED((np_rows, 16), F32),
            pltpu.SemaphoreType.DMA,
        ],
    )


def _swish(x):
    return x / (1.0 + jnp.exp(-x))


def _sinpi(a):
    n = jnp.floor(a)
    r = a - n
    s = jnp.minimum(r, 1.0 - r)
    half = n * 0.5
    sgn = 1.0 - 4.0 * (half - jnp.floor(half))
    t = s * jnp.pi
    q = t * t
    p = t * (1.0 + q * (-1.0 / 6.0 + q * (1.0 / 120.0
             + q * (-1.0 / 5040.0 + q * (1.0 / 362880.0)))))
    return sgn * p


def _edge_math_t(ps, pr, fm, w0, b0c, w1, w2, wup, valid):
    dn_t = (((0,), (0,)), ((), ()))
    vec = pr - ps
    len2 = jnp.sum(vec * vec, axis=0, keepdims=True)
    length = jnp.sqrt(len2)
    u = length / CUTOFF
    safe = jnp.where(u == 0.0, 1.0, u)
    k = (lax.broadcasted_iota(I32, (N_BASIS, 1), 0) + 1).astype(F32)
    bess = jnp.sqrt(2.0) * _sinpi(u * k) / safe
    u5 = u * u * u * u * u
    env = 1.0 - 21.0 * u5 + 35.0 * u5 * u - 15.0 * u5 * u * u
    env = jnp.where(u < 1.0, env, 0.0)
    basis = jnp.where(length == 0.0, 0.0, bess * env)

    h = _swish(lax.dot_general(w0, basis, dn_t,
                               preferred_element_type=F32) + b0c)
    h = _swish(lax.dot_general(w1, h, dn_t, preferred_element_type=F32))
    mix = lax.dot_general(w2, h, dn_t, preferred_element_type=F32)

    nh = vec / (length + 1e-9)
    m0 = fm[0:3] * wup[0] + fm[3:6] * wup[2]
    m1 = fm[0:3] * wup[1] + fm[3:6] * wup[3]
    d0 = jnp.sum(m0 * nh, axis=0, keepdims=True)
    d1 = jnp.sum(m1 * nh, axis=0, keepdims=True)
    sqrt3 = jnp.sqrt(3.0)
    sqrt75 = jnp.sqrt(7.5)
    s0 = sqrt3 * d0 * mix[0:1]
    s1 = sqrt3 * d1 * mix[1:2]
    vm0 = m0 * mix[2:3]
    vm1 = m1 * mix[3:4]
    t0 = sqrt75 * (nh * d0 - m0 * (1.0 / 3.0)) * mix[4:5]
    t1 = sqrt75 * (nh * d1 - m1 * (1.0 / 3.0)) * mix[5:6]
    zz = jnp.zeros((2, s0.shape[1]), F32)
    out = jnp.concatenate([s0, s1, vm0, vm1, t0, t1, zz], axis=0)
    return jnp.where(valid, out, 0.0)


def _node_math(agg_a, agg_b, fp, wdv, wds, wsc):
    a = [agg_a[:, 2:5], agg_a[:, 5:8], agg_b[:, 0:3], agg_b[:, 3:6]]
    f0, f1 = fp[:, 0:3], fp[:, 3:6]
    g = []
    for kk in range(2):
        v = (a[0] * wdv[0][kk] + a[1] * wdv[1][kk]
             + a[2] * wdv[2][kk] + a[3] * wdv[3][kk]
             + f0 * wsc[0][kk] + f1 * wsc[1][kk])
        s = agg_a[:, 0:1] * wds[0][kk] + agg_a[:, 1:2] * wds[1][kk]
        g.append(v * _swish(s))
    zz = jnp.zeros_like(g[0][:, 0:1])
    gated = jnp.concatenate([g[0], g[1], zz, zz], axis=1)
    ss0 = jnp.sum(g[0] * g[0])
    ss1 = jnp.sum(g[1] * g[1])
    return gated, (ss0, ss1)


BE = 4096
BN = 2000


def _edge_kernel_call(posT, fmT, w0, b0c, w1, w2, wup22, e_real, es_pad):
    nblk = es_pad // BE

    def body(wup_ref, ps_ref, pr_ref, fm_ref, w0_ref, b0_ref, w1_ref, w2_ref,
             out_ref):
        i = pl.program_id(0)
        cols = i * BE + lax.broadcasted_iota(I32, (1, BE), 1)
        valid = cols < e_real
        wup = (wup_ref[0, 0], wup_ref[0, 1], wup_ref[1, 0], wup_ref[1, 1])
        out_ref[...] = _edge_math_t(ps_ref[...], pr_ref[...], fm_ref[...],
                                    w0_ref[...], b0_ref[...], w1_ref[...],
                                    w2_ref[...], wup, valid)

    return pl.pallas_call(
        body,
        grid=(nblk,),
        in_specs=[
            pl.BlockSpec(memory_space=pltpu.SMEM),
            pl.BlockSpec((3, BE), lambda i: (0, i)),
            pl.BlockSpec((3, BE), lambda i, off=nblk: (0, i + off)),
            pl.BlockSpec((6, BE), lambda i: (0, i)),
            pl.BlockSpec((8, 64), lambda i: (0, 0)),
            pl.BlockSpec((64, 1), lambda i: (0, 0)),
            pl.BlockSpec((64, 64), lambda i: (0, 0)),
            pl.BlockSpec((64, 8), lambda i: (0, 0)),
        ],
        out_specs=pl.BlockSpec((16, BE), lambda i: (0, i)),
        out_shape=jax.ShapeDtypeStruct((16, es_pad), F32),
    )(wup22, posT, posT, fmT, w0, b0c, w1, w2)


def _node_kernel_call(acc, featp, wdv42, wds22, wsc22, n_nodes):
    nblk = n_nodes // BN

    def body(wdv_ref, wds_ref, wsc_ref, acc_ref, fp_ref, g_ref,
             ssq_ref):
        i = pl.program_id(0)
        agg = acc_ref[0] + acc_ref[1]
        agg_a = agg[:, 0:8]
        agg_b = agg[:, 8:16]
        wdv = tuple((wdv_ref[r, 0], wdv_ref[r, 1]) for r in range(4))
        wds = tuple((wds_ref[r, 0], wds_ref[r, 1]) for r in range(2))
        wsc = tuple((wsc_ref[r, 0], wsc_ref[r, 1]) for r in range(2))
        gated, (ss0, ss1) = _node_math(agg_a, agg_b, fp_ref[...], wdv, wds, wsc)
        g_ref[...] = gated
        row = jnp.concatenate(
            [ss0.reshape(1, 1), ss1.reshape(1, 1), jnp.zeros((1, 6), F32)],
            axis=1)

        @pl.when(i == 0)
        def _():
            ssq_ref[...] = row

        @pl.when(i != 0)
        def _():
            ssq_ref[...] = ssq_ref[...] + row

    return pl.pallas_call(
        body,
        grid=(nblk,),
        in_specs=[
            pl.BlockSpec(memory_space=pltpu.SMEM),
            pl.BlockSpec(memory_space=pltpu.SMEM),
            pl.BlockSpec(memory_space=pltpu.SMEM),
            pl.BlockSpec((2, BN, 16), lambda i: (0, i, 0)),
            pl.BlockSpec((BN, 8), lambda i: (i, 0)),
        ],
        out_specs=[
            pl.BlockSpec((BN, 8), lambda i: (i, 0)),
            pl.BlockSpec((1, 8), lambda i: (0, 0)),
        ],
        out_shape=[
            jax.ShapeDtypeStruct((n_nodes, 8), F32),
            jax.ShapeDtypeStruct((1, 8), F32),
        ],
    )(wdv42, wds22, wsc22, acc, featp)


def _final_kernel_call(nodes, gated, irms12, n_nodes):
    nblk = n_nodes // BN

    def body(ir_ref, nd_ref, g_ref, out_ref):
        disp = nd_ref[:, 0:3] + g_ref[:, 0:3] * ir_ref[0, 0]
        vel = g_ref[:, 3:6] * ir_ref[0, 1]
        out_ref[...] = jnp.concatenate([disp, vel], axis=1)

    return pl.pallas_call(
        body,
        grid=(nblk,),
        in_specs=[
            pl.BlockSpec(memory_space=pltpu.SMEM),
            pl.BlockSpec((BN, 6), lambda i: (i, 0)),
            pl.BlockSpec((BN, 8), lambda i: (i, 0)),
        ],
        out_specs=pl.BlockSpec((BN, 6), lambda i: (i, 0)),
        out_shape=jax.ShapeDtypeStruct((n_nodes, 6), F32),
    )(irms12, nodes, gated)


def _prep_weights(p, m, gvec, rms_prev):
    inv_r = 1.0 / rms_prev
    wsc = jnp.zeros((2, 2), F32).at[:m].set(p['W_sc'] * inv_r[:m, None])
    wup = jnp.zeros((2, 2), F32).at[:m, :m].set(p['W_up'] * inv_r[:m, None])
    w0 = p['W0'][:N_BASIS]
    b0 = (gvec.reshape(1, -1) @ p['W0'][N_BASIS:]).astype(F32).reshape(64, 1)
    w1 = p['W1']
    w2 = jnp.zeros((64, 8), F32)
    if m == 2:
        w2 = w2.at[:, 0:6].set(p['W2'])
    else:
        w2 = (w2.at[:, 0].set(p['W2'][:, 0])
                .at[:, 2].set(p['W2'][:, 1])
                .at[:, 4].set(p['W2'][:, 2]))
    inv_an = 1.0 / jnp.sqrt(AVG_NEIGH)
    wdvs = p['Wd_v'] * inv_an
    if m == 2:
        wdv = wdvs
    else:
        wdv = jnp.zeros((4, 2), F32).at[0].set(wdvs[0]).at[2].set(wdvs[1])
    wds = jnp.zeros((2, 2), F32).at[:m].set(p['Wd_s'] * inv_an)
    return wsc, wup, w0, b0, w1, w2, wdv, wds


def kernel(nodes, globals, params, senders, receivers):
    n = nodes.shape[0]
    e = senders.shape[0]
    assert n % BN == 0

    unit = NW * CHR * LANE
    es = ((e + unit - 1) // unit) * unit
    np_rows = ((n + NS * 8 - 1) // (NS * 8)) * (NS * 8)

    pad = jnp.zeros((es - e,), I32)
    send_p = jnp.concatenate([senders, pad])
    recv_p = jnp.concatenate([receivers, pad])
    sr2 = jnp.concatenate([send_p, recv_p]).reshape(-1, LANE)
    send2 = send_p.reshape(-1, LANE)
    recv2 = recv_p.reshape(-1, LANE)

    pos_tab = jnp.pad(nodes[:, 0:3], ((0, 0), (0, 5)))
    feat = jnp.pad(nodes[:, 3:6], ((0, 0), (0, 5)))
    zeros_acc = jnp.zeros((np_rows, 16), F32)
    gvec = globals.astype(F32)

    posg = _make_gather(2 * es // LANE, n)(pos_tab, sr2)
    posT = jnp.transpose(posg[:, 0:3])

    rms_prev = jnp.ones((2,), F32)
    for li, p in enumerate(params):
        m = 1 if li == 0 else 2
        wsc, wup, w0, b0, w1, w2, wdv, wds = _prep_weights(p, m, gvec, rms_prev)
        fmsg = _make_gather(es // LANE, n)(feat, send2)
        fmT = jnp.transpose(fmsg[:, 0:6])
        eoutT = _edge_kernel_call(posT, fmT, w0, b0, w1, w2, wup, e, es)
        eout = jnp.transpose(eoutT)
        acc = _make_scatter(es // LANE, np_rows)(eout, recv2, zeros_acc)
        gated, ssq = _node_kernel_call(acc, feat, wdv, wds, wsc, n)
        rms_prev = jnp.sqrt(ssq[0, 0:2] / n) + 1e-5
        feat = gated

    irms = (1.0 / rms_prev).reshape(1, 2)
    return _final_kernel_call(nodes, feat, irms, n)


# speedup vs baseline: 1.1107x; 1.0592x over previous
import functools

import jax
import jax.numpy as jnp
from jax import lax
from jax.experimental import pallas as pl
from jax.experimental.pallas import tpu as pltpu
from jax.experimental.pallas import tpu_sc as plsc

N_BASIS = 8
CUTOFF = 10.0
AVG_NEIGH = 16.0

NC = 2
NS = 16
NW = NC * NS
LANE = 128
CHR = 40

F32 = jnp.float32
I32 = jnp.int32


def _mesh():
    return plsc.VectorSubcoreMesh(core_axis_name="c", subcore_axis_name="s")


@functools.cache
def _make_gather(total_rows: int, table_rows: int):
    n_chunks = total_rows // (NW * CHR)
    assert total_rows == NW * CHR * n_chunks
    ce = CHR * LANE

    def body(table, idx2d, out, idxbuf, ebuf, sem):
        wid = lax.axis_index("s") * NC + lax.axis_index("c")
        for ch in range(n_chunks):
            row0 = wid * (CHR * n_chunks) + ch * CHR
            pltpu.sync_copy(idx2d.at[pl.ds(row0, CHR)], idxbuf)

            def fire(j, carry):
                pltpu.async_copy(table.at[idxbuf.at[j]],
                                 ebuf.at[pl.ds(j * LANE, LANE)], sem)
                return carry

            lax.fori_loop(0, CHR, fire, 0)
            pltpu.make_async_copy(table.at[pl.ds(0, ce)], ebuf, sem).wait()
            pltpu.sync_copy(ebuf, out.at[pl.ds(row0 * LANE, ce)])

    return pl.kernel(
        body,
        out_type=jax.ShapeDtypeStruct((total_rows * LANE, 8), F32),
        mesh=_mesh(),
        compiler_params=pltpu.CompilerParams(use_tc_tiling_on_sc=False),
        scratch_types=[
            pltpu.VMEM((CHR, LANE), I32),
            pltpu.VMEM((ce, 8), F32),
            pltpu.SemaphoreType.DMA,
        ],
    )


CHS = 20


@functools.cache
def _make_scatter(total_rows: int, np_rows: int):
    n_chunks = total_rows // (NW * CHS)
    assert total_rows == NW * CHS * n_chunks
    ce = CHS * LANE
    rz = np_rows // NS
    assert rz * NS == np_rows and rz % 8 == 0

    def body(edge, idx2d, zeros_hbm, out, idxbuf, ebuf, acc, sem):
        c = lax.axis_index("c")
        s = lax.axis_index("s")
        wid = s * NC + c
        pltpu.sync_copy(zeros_hbm.at[pl.ds(s * rz, rz)], acc.at[pl.ds(s * rz, rz)])
        plsc.subcore_barrier()
        for ch in range(n_chunks):
            row0 = wid * (CHS * n_chunks) + ch * CHS
            pltpu.sync_copy(idx2d.at[pl.ds(row0, CHS)], idxbuf)
            pltpu.sync_copy(edge.at[pl.ds(row0 * LANE, ce)], ebuf)

            def fire(j, carry):
                pltpu.async_copy(ebuf.at[pl.ds(j * LANE, LANE)],
                                 acc.at[idxbuf.at[j]], sem, add=True)
                return carry

            lax.fori_loop(0, CHS, fire, 0)
            pltpu.make_async_copy(edge.at[pl.ds(0, ce)], ebuf, sem).wait()
        plsc.subcore_barrier()
        pltpu.sync_copy(acc.at[pl.ds(s * rz, rz)], out.at[c, pl.ds(s * rz, rz)])

    return pl.kernel(
        body,
        out_type=jax.ShapeDtypeStruct((NC, np_rows, 16), F32),
        mesh=_mesh(),
        compiler_params=pltpu.CompilerParams(use_tc_tiling_on_sc=False),
        scratch_types=[
            pltpu.VMEM((CHS, LANE), I32),
            pltpu.VMEM((ce, 16), F32),
            pltpu.VMEM_SHARED((np_rows, 16), F32),
            pltpu.SemaphoreType.DMA,
        ],
    )


def _swish(x):
    return x / (1.0 + jnp.exp(-x))


def _sinpi(a):
    n = jnp.floor(a)
    r = a - n
    s = jnp.minimum(r, 1.0 - r)
    half = n * 0.5
    sgn = 1.0 - 4.0 * (half - jnp.floor(half))
    t = s * jnp.pi
    q = t * t
    p = t * (1.0 + q * (-1.0 / 6.0 + q * (1.0 / 120.0
             + q * (-1.0 / 5040.0 + q * (1.0 / 362880.0)))))
    return sgn * p


def _edge_math_t(ps, pr, fm, w0, b0c, w1, w2, wup, valid):
    dn_t = (((0,), (0,)), ((), ()))
    vec = pr - ps
    len2 = jnp.sum(vec * vec, axis=0, keepdims=True)
    length = jnp.sqrt(len2)
    u = length / CUTOFF
    safe = jnp.where(u == 0.0, 1.0, u)
    k = (lax.broadcasted_iota(I32, (N_BASIS, 1), 0) + 1).astype(F32)
    bess = jnp.sqrt(2.0) * _sinpi(u * k) / safe
    u5 = u * u * u * u * u
    env = 1.0 - 21.0 * u5 + 35.0 * u5 * u - 15.0 * u5 * u * u
    env = jnp.where(u < 1.0, env, 0.0)
    basis = jnp.where(length == 0.0, 0.0, bess * env)

    h = _swish(lax.dot_general(w0, basis, dn_t,
                               preferred_element_type=F32) + b0c)
    h = _swish(lax.dot_general(w1, h, dn_t, preferred_element_type=F32))
    mix = lax.dot_general(w2, h, dn_t, preferred_element_type=F32)

    nh = vec / (length + 1e-9)
    m0 = fm[0:3] * wup[0] + fm[3:6] * wup[2]
    m1 = fm[0:3] * wup[1] + fm[3:6] * wup[3]
    d0 = jnp.sum(m0 * nh, axis=0, keepdims=True)
    d1 = jnp.sum(m1 * nh, axis=0, keepdims=True)
    sqrt3 = jnp.sqrt(3.0)
    sqrt75 = jnp.sqrt(7.5)
    s0 = sqrt3 * d0 * mix[0:1]
    s1 = sqrt3 * d1 * mix[1:2]
    vm0 = m0 * mix[2:3]
    vm1 = m1 * mix[3:4]
    t0 = sqrt75 * (nh * d0 - m0 * (1.0 / 3.0)) * mix[4:5]
    t1 = sqrt75 * (nh * d1 - m1 * (1.0 / 3.0)) * mix[5:6]
    zz = jnp.zeros((2, s0.shape[1]), F32)
    out = jnp.concatenate([s0, s1, vm0, vm1, t0, t1, zz], axis=0)
    return jnp.where(valid, out, 0.0)


def _node_math(agg_a, agg_b, fp, wdv, wds, wsc):
    a = [agg_a[:, 2:5], agg_a[:, 5:8], agg_b[:, 0:3], agg_b[:, 3:6]]
    f0, f1 = fp[:, 0:3], fp[:, 3:6]
    g = []
    for kk in range(2):
        v = (a[0] * wdv[0][kk] + a[1] * wdv[1][kk]
             + a[2] * wdv[2][kk] + a[3] * wdv[3][kk]
             + f0 * wsc[0][kk] + f1 * wsc[1][kk])
        s = agg_a[:, 0:1] * wds[0][kk] + agg_a[:, 1:2] * wds[1][kk]
        g.append(v * _swish(s))
    zz = jnp.zeros_like(g[0][:, 0:1])
    gated = jnp.concatenate([g[0], g[1], zz, zz], axis=1)
    ss0 = jnp.sum(g[0] * g[0])
    ss1 = jnp.sum(g[1] * g[1])
    return gated, (ss0, ss1)


BE = 4096
BN = 2000


def _edge_kernel_call(posT, fmT, w0, b0c, w1, w2, wup22, e_real, es_pad):
    nblk = es_pad // BE

    def body(wup_ref, ps_ref, pr_ref, fm_ref, w0_ref, b0_ref, w1_ref, w2_ref,
             out_ref):
        i = pl.program_id(0)
        cols = i * BE + lax.broadcasted_iota(I32, (1, BE), 1)
        valid = cols < e_real
        wup = (wup_ref[0, 0], wup_ref[0, 1], wup_ref[1, 0], wup_ref[1, 1])
        out_ref[...] = _edge_math_t(ps_ref[...], pr_ref[...], fm_ref[...],
                                    w0_ref[...], b0_ref[...], w1_ref[...],
                                    w2_ref[...], wup, valid)

    return pl.pallas_call(
        body,
        grid=(nblk,),
        in_specs=[
            pl.BlockSpec(memory_space=pltpu.SMEM),
            pl.BlockSpec((3, BE), lambda i: (0, i)),
            pl.BlockSpec((3, BE), lambda i, off=nblk: (0, i + off)),
            pl.BlockSpec((6, BE), lambda i: (0, i)),
            pl.BlockSpec((8, 64), lambda i: (0, 0)),
            pl.BlockSpec((64, 1), lambda i: (0, 0)),
            pl.BlockSpec((64, 64), lambda i: (0, 0)),
            pl.BlockSpec((64, 8), lambda i: (0, 0)),
        ],
        out_specs=pl.BlockSpec((16, BE), lambda i: (0, i)),
        out_shape=jax.ShapeDtypeStruct((16, es_pad), F32),
    )(wup22, posT, posT, fmT, w0, b0c, w1, w2)


def _gate_mats(wdv, wds, wsc):
    i3 = jnp.eye(3, dtype=F32)
    o3 = jnp.ones((1, 3), F32)
    z = jnp.zeros
    wv = jnp.concatenate([
        z((2, 6), F32),
        jnp.kron(wdv[0].reshape(1, 2), i3),
        jnp.kron(wdv[1].reshape(1, 2), i3),
        jnp.kron(wdv[2].reshape(1, 2), i3),
        jnp.kron(wdv[3].reshape(1, 2), i3),
        z((2, 6), F32),
        jnp.kron(wsc[0].reshape(1, 2), i3),
        jnp.kron(wsc[1].reshape(1, 2), i3),
        z((2, 6), F32),
    ], axis=0)
    ws = jnp.concatenate([
        jnp.kron(wds[0].reshape(1, 2), o3),
        jnp.kron(wds[1].reshape(1, 2), o3),
        z((22, 6), F32),
    ], axis=0)
    pad = z((24, 2), F32)
    return (jnp.concatenate([wv, pad], axis=1),
            jnp.concatenate([ws, pad], axis=1))


def _node_kernel_call(acc, featp, wv248, ws248, n_nodes):
    nblk = n_nodes // BN

    def body(acc_ref, fp_ref, wv_ref, ws_ref, g_ref, ssq_ref):
        i = pl.program_id(0)
        agg = acc_ref[0] + acc_ref[1]
        x = jnp.concatenate([agg, fp_ref[...]], axis=1)
        gv = jnp.dot(x, wv_ref[...], preferred_element_type=F32)
        gs = jnp.dot(x, ws_ref[...], preferred_element_type=F32)
        gated = gv * _swish(gs)
        g_ref[...] = gated
        row = jnp.sum(gated * gated, axis=0, keepdims=True)

        @pl.when(i == 0)
        def _():
            ssq_ref[...] = row

        @pl.when(i != 0)
        def _():
            ssq_ref[...] = ssq_ref[...] + row

    return pl.pallas_call(
        body,
        grid=(nblk,),
        in_specs=[
            pl.BlockSpec((2, BN, 16), lambda i: (0, i, 0)),
            pl.BlockSpec((BN, 8), lambda i: (i, 0)),
            pl.BlockSpec((24, 8), lambda i: (0, 0)),
            pl.BlockSpec((24, 8), lambda i: (0, 0)),
        ],
        out_specs=[
            pl.BlockSpec((BN, 8), lambda i: (i, 0)),
            pl.BlockSpec((1, 8), lambda i: (0, 0)),
        ],
        out_shape=[
            jax.ShapeDtypeStruct((n_nodes, 8), F32),
            jax.ShapeDtypeStruct((1, 8), F32),
        ],
    )(acc, featp, wv248, ws248)


def _final_kernel_call(nodes, gated, irms12, n_nodes):
    nblk = n_nodes // BN

    def body(ir_ref, nd_ref, g_ref, out_ref):
        disp = nd_ref[:, 0:3] + g_ref[:, 0:3] * ir_ref[0, 0]
        vel = g_ref[:, 3:6] * ir_ref[0, 1]
        out_ref[...] = jnp.concatenate([disp, vel], axis=1)

    return pl.pallas_call(
        body,
        grid=(nblk,),
        in_specs=[
            pl.BlockSpec(memory_space=pltpu.SMEM),
            pl.BlockSpec((BN, 6), lambda i: (i, 0)),
            pl.BlockSpec((BN, 8), lambda i: (i, 0)),
        ],
        out_specs=pl.BlockSpec((BN, 6), lambda i: (i, 0)),
        out_shape=jax.ShapeDtypeStruct((n_nodes, 6), F32),
    )(irms12, nodes, gated)


def _prep_weights(p, m, gvec, rms_prev):
    inv_r = 1.0 / rms_prev
    wsc = jnp.zeros((2, 2), F32).at[:m].set(p['W_sc'] * inv_r[:m, None])
    wup = jnp.zeros((2, 2), F32).at[:m, :m].set(p['W_up'] * inv_r[:m, None])
    w0 = p['W0'][:N_BASIS]
    b0 = (gvec.reshape(1, -1) @ p['W0'][N_BASIS:]).astype(F32).reshape(64, 1)
    w1 = p['W1']
    w2 = jnp.zeros((64, 8), F32)
    if m == 2:
        w2 = w2.at[:, 0:6].set(p['W2'])
    else:
        w2 = (w2.at[:, 0].set(p['W2'][:, 0])
                .at[:, 2].set(p['W2'][:, 1])
                .at[:, 4].set(p['W2'][:, 2]))
    inv_an = 1.0 / jnp.sqrt(AVG_NEIGH)
    wdvs = p['Wd_v'] * inv_an
    if m == 2:
        wdv = wdvs
    else:
        wdv = jnp.zeros((4, 2), F32).at[0].set(wdvs[0]).at[2].set(wdvs[1])
    wds = jnp.zeros((2, 2), F32).at[:m].set(p['Wd_s'] * inv_an)
    return wsc, wup, w0, b0, w1, w2, wdv, wds


def kernel(nodes, globals, params, senders, receivers):
    n = nodes.shape[0]
    e = senders.shape[0]
    assert n % BN == 0

    unit = NW * CHR * LANE
    es = ((e + unit - 1) // unit) * unit
    np_rows = ((n + NS * 8 - 1) // (NS * 8)) * (NS * 8)

    pad = jnp.zeros((es - e,), I32)
    send_p = jnp.concatenate([senders, pad])
    recv_p = jnp.concatenate([receivers, pad])
    sr2 = jnp.concatenate([send_p, recv_p]).reshape(-1, LANE)
    send2 = send_p.reshape(-1, LANE)
    recv2 = recv_p.reshape(-1, LANE)

    pos_tab = jnp.pad(nodes[:, 0:3], ((0, 0), (0, 5)))
    feat = jnp.pad(nodes[:, 3:6], ((0, 0), (0, 5)))
    zeros_acc = jnp.zeros((np_rows, 16), F32)
    gvec = globals.astype(F32)

    posg = _make_gather(2 * es // LANE, n)(pos_tab, sr2)
    posT = jnp.transpose(posg[:, 0:3])

    rms_prev = jnp.ones((2,), F32)
    for li, p in enumerate(params):
        m = 1 if li == 0 else 2
        wsc, wup, w0, b0, w1, w2, wdv, wds = _prep_weights(p, m, gvec, rms_prev)
        fmsg = _make_gather(es // LANE, n)(feat, send2)
        fmT = jnp.transpose(fmsg[:, 0:6])
        eoutT = _edge_kernel_call(posT, fmT, w0, b0, w1, w2, wup, e, es)
        eout = jnp.transpose(eoutT)
        acc = _make_scatter(es // LANE, np_rows)(eout, recv2, zeros_acc)
        wv, ws = _gate_mats(wdv, wds, wsc)
        gated, ssq = _node_kernel_call(acc, feat, wv, ws, n)
        ss2 = jnp.stack([jnp.sum(ssq[0, 0:3]), jnp.sum(ssq[0, 3:6])])
        rms_prev = jnp.sqrt(ss2 / n) + 1e-5
        feat = gated

    irms = (1.0 / rms_prev).reshape(1, 2)
    return _final_kernel_call(nodes, feat, irms, n)
